# baseline (speedup 1.0000x reference)
import functools

import jax
import jax.numpy as jnp
from jax import lax
from jax.experimental import pallas as pl
from jax.experimental.pallas import tpu as pltpu
from jax.experimental.pallas import tpu_sc as plsc

NUM_BUCKETS = 32
NUM_HEADS = 16
Q_LEN = 2048
K_LEN = 2048
LV = 4096
LG = 4096
ROWS = 128
BLK_I = 1024
PER_TILE = 128

_THRESH = (19, 21, 24, 27, 31, 35, 40, 46, 52, 59, 67, 77, 87, 99, 113)


def _dyn_gather16(x, idx):
    dnums = lax.GatherDimensionNumbers(
        offset_dims=(), collapsed_slice_dims=(0,), start_index_map=(0,)
    )
    return lax.gather(
        x,
        idx[:, None],
        dnums,
        slice_sizes=(1,),
        mode=lax.GatherScatterMode.PROMISE_IN_BOUNDS,
    )


def _lookup_sc(table_hbm, out_hbm, table_v, vbuf):
    wid = lax.axis_index("s") * 2 + lax.axis_index("c")
    pltpu.sync_copy(table_hbm, table_v)
    base = wid * PER_TILE
    lane = lax.iota(jnp.int32, 16)
    for grp in range(PER_TILE // 16):
        zero = jnp.zeros((16,), jnp.int32)
        one = jnp.full((16,), 1, jnp.int32)
        m = lane + jnp.broadcast_to(base + grp * 16, (16,))
        rel = jnp.maximum(jnp.broadcast_to(Q_LEN - 1, (16,)) - m, zero)
        large = jnp.full((16,), 16, jnp.int32)
        for t in _THRESH:
            tv = jnp.full((16,), t, jnp.int32)
            large = large + jnp.where(rel >= tv, one, zero)
        b = jnp.where(rel < jnp.full((16,), 16, jnp.int32), rel, large)
        blo = jnp.minimum(b, jnp.full((16,), 15, jnp.int32))
        bhi = jnp.maximum(b - jnp.full((16,), 16, jnp.int32), zero)
        is_lo = b < jnp.full((16,), 16, jnp.int32)
        for j in range(NUM_HEADS):
            tlo = table_v[pl.ds(j * NUM_BUCKETS, 16)]
            thi = table_v[pl.ds(j * NUM_BUCKETS + 16, 16)]
            col = jnp.where(is_lo, _dyn_gather16(tlo, blo), _dyn_gather16(thi, bhi))
            vbuf[j, pl.ds(grp * 16, 16)] = col
    pltpu.sync_copy(vbuf, out_hbm.at[:, pl.ds(base, PER_TILE)])


@functools.partial(
    pl.kernel,
    out_type=jax.ShapeDtypeStruct((NUM_HEADS, LV), jnp.float32),
    mesh=plsc.VectorSubcoreMesh(core_axis_name="c", subcore_axis_name="s"),
    scratch_types=[
        pltpu.VMEM((NUM_HEADS * NUM_BUCKETS,), jnp.float32),
        pltpu.VMEM((NUM_HEADS, PER_TILE), jnp.float32),
    ],
)
def _lookup(table_hbm, out_hbm, table_v, vbuf):
    _lookup_sc(table_hbm, out_hbm, table_v, vbuf)


def _expand_kernel(v_ref, out_ref, g_ref):
    g = pl.program_id(1)

    @pl.when(g == 0)
    def _build():
        v = v_ref[...].reshape(1, LV)
        wide = jnp.broadcast_to(v, (ROWS, LV))
        shifted = pltpu.roll(wide, LV - 127, 1, stride=1, stride_axis=0)
        g_ref[...] = shifted[:, :LG]

    ngrp = BLK_I // ROWS
    for k in range(ngrp):
        out_ref[0, 0, ROWS * k : ROWS * (k + 1), :] = g_ref[
            :, pl.ds(128 * (15 - ngrp * g - k), K_LEN)
        ]


def kernel(query_length, key_length, relative_attention_bias):
    del query_length, key_length
    table_flat = relative_attention_bias.T.reshape(-1)
    v = _lookup(table_flat)
    v3 = v.reshape(NUM_HEADS, 1, LV)
    out = pl.pallas_call(
        _expand_kernel,
        grid=(NUM_HEADS, Q_LEN // BLK_I),
        in_specs=[
            pl.BlockSpec((1, 1, LV), lambda h, g: (h, 0, 0)),
        ],
        out_specs=pl.BlockSpec(
            (1, 1, BLK_I, K_LEN), lambda h, g: (0, h, g, 0)
        ),
        out_shape=jax.ShapeDtypeStruct(
            (1, NUM_HEADS, Q_LEN, K_LEN), jnp.float32
        ),
        scratch_shapes=[pltpu.VMEM((ROWS, LG), jnp.float32)],
        compiler_params=pltpu.CompilerParams(
            dimension_semantics=("parallel", "arbitrary"),
        ),
    )(v3)
    return out


# speedup vs baseline: 1.0126x; 1.0126x over previous
import functools

import jax
import jax.numpy as jnp
from jax import lax
from jax.experimental import pallas as pl
from jax.experimental.pallas import tpu as pltpu
from jax.experimental.pallas import tpu_sc as plsc

NUM_BUCKETS = 32
NUM_HEADS = 16
Q_LEN = 2048
K_LEN = 2048
LV = 4096
LG = 4096
ROWS = 128
BLK_I = 1024
PER_TILE = 128

_THRESH = (19, 21, 24, 27, 31, 35, 40, 46, 52, 59, 67, 77, 87, 99, 113)


def _dyn_gather16(x, idx):
    dnums = lax.GatherDimensionNumbers(
        offset_dims=(), collapsed_slice_dims=(0,), start_index_map=(0,)
    )
    return lax.gather(
        x,
        idx[:, None],
        dnums,
        slice_sizes=(1,),
        mode=lax.GatherScatterMode.PROMISE_IN_BOUNDS,
    )


def _lookup_sc(table_hbm, out_hbm, table_v, vbuf):
    wid = lax.axis_index("s") * 2 + lax.axis_index("c")
    pltpu.sync_copy(table_hbm, table_v)
    base = wid * PER_TILE
    lane = lax.iota(jnp.int32, 16)
    for grp in range(PER_TILE // 16):
        zero = jnp.zeros((16,), jnp.int32)
        one = jnp.full((16,), 1, jnp.int32)
        m = lane + jnp.broadcast_to(base + grp * 16, (16,))
        rel = jnp.maximum(jnp.broadcast_to(Q_LEN - 1, (16,)) - m, zero)
        large = jnp.full((16,), 16, jnp.int32)
        for t in _THRESH:
            tv = jnp.full((16,), t, jnp.int32)
            large = large + jnp.where(rel >= tv, one, zero)
        b = jnp.where(rel < jnp.full((16,), 16, jnp.int32), rel, large)
        blo = jnp.minimum(b, jnp.full((16,), 15, jnp.int32))
        bhi = jnp.maximum(b - jnp.full((16,), 16, jnp.int32), zero)
        is_lo = b < jnp.full((16,), 16, jnp.int32)
        for j in range(NUM_HEADS):
            tlo = table_v[pl.ds(j * NUM_BUCKETS, 16)]
            thi = table_v[pl.ds(j * NUM_BUCKETS + 16, 16)]
            col = jnp.where(is_lo, _dyn_gather16(tlo, blo), _dyn_gather16(thi, bhi))
            vbuf[j, pl.ds(grp * 16, 16)] = col
    pltpu.sync_copy(vbuf, out_hbm.at[:, pl.ds(base, PER_TILE)])


@functools.partial(
    pl.kernel,
    out_type=jax.ShapeDtypeStruct((NUM_HEADS, LV), jnp.float32),
    mesh=plsc.VectorSubcoreMesh(core_axis_name="c", subcore_axis_name="s"),
    scratch_types=[
        pltpu.VMEM((NUM_HEADS * NUM_BUCKETS,), jnp.float32),
        pltpu.VMEM((NUM_HEADS, PER_TILE), jnp.float32),
    ],
)
def _lookup(table_hbm, out_hbm, table_v, vbuf):
    _lookup_sc(table_hbm, out_hbm, table_v, vbuf)


def _expand_kernel(v_ref, out_ref, g_ref):
    h = pl.program_id(0)
    g = pl.program_id(1)

    @pl.when(g == 0)
    def _build():
        vh = pltpu.roll(v_ref[...], NUM_HEADS - h, 0)[0:1, :]
        wide = jnp.broadcast_to(vh, (ROWS, LV))
        shifted = pltpu.roll(wide, LV - 127, 1, stride=1, stride_axis=0)
        g_ref[...] = shifted[:, :LG]

    ngrp = BLK_I // ROWS
    for k in range(ngrp):
        out_ref[0, 0, ROWS * k : ROWS * (k + 1), :] = g_ref[
            :, pl.ds(128 * (15 - ngrp * g - k), K_LEN)
        ]


def kernel(query_length, key_length, relative_attention_bias):
    del query_length, key_length
    table_flat = relative_attention_bias.T.reshape(-1)
    v = _lookup(table_flat)
    out = pl.pallas_call(
        _expand_kernel,
        grid=(NUM_HEADS, Q_LEN // BLK_I),
        in_specs=[
            pl.BlockSpec((NUM_HEADS, LV), lambda h, g: (0, 0)),
        ],
        out_specs=pl.BlockSpec(
            (1, 1, BLK_I, K_LEN), lambda h, g: (0, h, g, 0)
        ),
        out_shape=jax.ShapeDtypeStruct(
            (1, NUM_HEADS, Q_LEN, K_LEN), jnp.float32
        ),
        scratch_shapes=[pltpu.VMEM((ROWS, LG), jnp.float32)],
        compiler_params=pltpu.CompilerParams(
            dimension_semantics=("parallel", "arbitrary"),
        ),
    )(v)
    return out
